# baseline (speedup 1.0000x reference)
import jax
import jax.numpy as jnp
from jax import lax
from jax.experimental import pallas as pl
from jax.experimental.pallas import tpu as pltpu
from jax.experimental.pallas import tpu_sc as plsc

NUM_WORKERS = 32
CHUNK_ROWS = 4
LANES = 16
UNROLL = 8
TRASH = 2048


def _build(B, S, D, n_table):
    assert B % NUM_WORKERS == 0
    rows_per_w = B // NUM_WORKERS
    assert rows_per_w % (2 * CHUNK_ROWS) == 0
    n_chunks = rows_per_w // CHUNK_ROWS
    CS = CHUNK_ROWS * S
    assert D % LANES == 0
    d_regs = D // LANES
    max_comp = CHUNK_ROWS * (-(-S // UNROLL) * UNROLL)
    assert max_comp <= TRASH
    n_slices = -(-max_comp // 128)
    rows_cap = n_slices * 128
    xc_cap = rows_cap + 128
    xcs_cap = TRASH + LANES

    mesh = plsc.VectorSubcoreMesh(core_axis_name="c", subcore_axis_name="s")

    def body(x_hbm, d_hbm, p_hbm, table_hbm, out_hbm,
             xv0, dv0, xc0, rows0, pvb0, xv1, dv1, xc1, rows1, pvb1,
             xcs0, xcs1, z0v, outb, gsem0, gsem1, csem0, csem1):
        wid = lax.axis_index("s") * 2 + lax.axis_index("c")
        sid = lax.axis_index("s")
        bufs = (
            (xv0, dv0, xc0, rows0, pvb0, gsem0, csem0, xcs0),
            (xv1, dv1, xc1, rows1, pvb1, gsem1, csem1, xcs1),
        )

        pltpu.sync_copy(table_hbm.at[pl.ds(0, 8)], z0v)
        slack_fill = wid * LANES + lax.iota(jnp.int32, LANES)

        def compact_dma(buf):
            xv, dv, csem, xcs = buf[0], buf[1], buf[6], buf[7]
            return pltpu.make_async_copy(xv, xcs.at[sid].at[dv], csem)

        def read_meta(buf):
            pvb = buf[4]
            meta = []
            for r in range(CHUNK_ROWS):
                pvr = pvb[pl.ds(r * LANES, LANES)]
                meta.append((pvr[0], pvr[1], pvr[2]))
            return meta

        def load_compact_start(c, buf):
            xv, dv, _, _, pvb = buf[0], buf[1], buf[2], buf[3], buf[4]
            row0 = wid * rows_per_w + c * CHUNK_ROWS
            pltpu.sync_copy(x_hbm.at[pl.ds(row0 * S, CS)], xv)
            pltpu.sync_copy(d_hbm.at[pl.ds(row0 * S, CS)], dv)
            pltpu.sync_copy(
                p_hbm.at[pl.ds(row0 * LANES, CHUNK_ROWS * LANES)], pvb
            )
            compact_dma(buf).start()

        def wait_compact(buf):
            xc, pvb, xcs = buf[2], buf[4], buf[7]
            compact_dma(buf).wait()
            pltpu.sync_copy(
                xcs.at[sid].at[pl.ds(0, rows_cap)], xc.at[pl.ds(0, rows_cap)]
            )
            hi_last = pvb[pl.ds((CHUNK_ROWS - 1) * LANES, LANES)][1]
            for k in range(128 // LANES):
                xc[pl.ds(hi_last + k * LANES, LANES)] = slack_fill

        def gather_copies(buf):
            xc, rows_v, pvb, gsem = buf[2], buf[3], buf[4], buf[5]
            total = pvb[pl.ds((CHUNK_ROWS - 1) * LANES, LANES)][1]
            for s in range(n_slices):
                yield (
                    s * 128 < total,
                    pltpu.make_async_copy(
                        table_hbm.at[xc.at[pl.ds(s * 128, 128)]],
                        rows_v.at[pl.ds(s * 128, 128)],
                        gsem,
                    ),
                )

        def fire(buf):
            for pred, cp in gather_copies(buf):
                @pl.when(pred)
                def _():
                    cp.start()

        def wait(buf):
            for pred, cp in gather_copies(buf):
                @pl.when(pred)
                def _():
                    cp.wait()

        def process(c, buf, meta):
            rows_v = buf[3]
            row0 = wid * rows_per_w + c * CHUNK_ROWS
            z0 = [z0v[0, pl.ds(d * LANES, LANES)] for d in range(d_regs)]
            z = jnp.zeros((LANES,), jnp.float32)
            for r in range(CHUNK_ROWS):
                lo, hi, cnt = meta[r]

                def blk(b, accs):
                    base = lo + b * UNROLL
                    out = list(accs)
                    for j in range(UNROLL):
                        for d in range(d_regs):
                            out[d] = out[d] + rows_v[base + j, pl.ds(d * LANES, LANES)]
                    return tuple(out)

                nb = lax.shift_right_logical(hi - lo, 3)
                accs = lax.fori_loop(0, nb, blk, (z,) * d_regs)
                cntf = jnp.full((LANES,), cnt, jnp.int32).astype(jnp.float32)
                padf = jnp.full(
                    (LANES,), hi - lo - cnt, jnp.int32
                ).astype(jnp.float32)
                inv = jnp.float32(1.0) / (cntf + jnp.float32(1e-9))
                for d in range(d_regs):
                    outb[r, pl.ds(d * LANES, LANES)] = (
                        accs[d] - padf * z0[d]
                    ) * inv
            pltpu.sync_copy(outb, out_hbm.at[pl.ds(row0, CHUNK_ROWS)])

        load_compact_start(0, bufs[0])
        wait_compact(bufs[0])
        fire(bufs[0])
        load_compact_start(1, bufs[1])

        last = jnp.int32(n_chunks - 1)

        def pair_body(i, carry):
            c0 = 2 * i
            wait_compact(bufs[1])
            fire(bufs[1])
            wait(bufs[0])
            m0 = read_meta(bufs[0])
            load_compact_start(jnp.minimum(c0 + 2, last), bufs[0])
            process(c0, bufs[0], m0)
            wait_compact(bufs[0])
            fire(bufs[0])
            wait(bufs[1])
            m1 = read_meta(bufs[1])
            load_compact_start(jnp.minimum(c0 + 3, last), bufs[1])
            process(c0 + 1, bufs[1], m1)
            return carry

        lax.fori_loop(0, n_chunks // 2, pair_body, 0)
        wait(bufs[0])
        wait_compact(bufs[1])

    return pl.kernel(
        body,
        out_type=jax.ShapeDtypeStruct((B, D), jnp.float32),
        mesh=mesh,
        compiler_params=pltpu.CompilerParams(
            use_tc_tiling_on_sc=False, needs_layout_passes=False
        ),
        scratch_types=[
            pltpu.VMEM((CS,), jnp.int32),
            pltpu.VMEM((CS,), jnp.int32),
            pltpu.VMEM((xc_cap,), jnp.int32),
            pltpu.VMEM((rows_cap, D), jnp.float32),
            pltpu.VMEM((CHUNK_ROWS * LANES,), jnp.int32),
            pltpu.VMEM((CS,), jnp.int32),
            pltpu.VMEM((CS,), jnp.int32),
            pltpu.VMEM((xc_cap,), jnp.int32),
            pltpu.VMEM((rows_cap, D), jnp.float32),
            pltpu.VMEM((CHUNK_ROWS * LANES,), jnp.int32),
            pltpu.VMEM_SHARED((16, xcs_cap), jnp.int32),
            pltpu.VMEM_SHARED((16, xcs_cap), jnp.int32),
            pltpu.VMEM((8, D), jnp.float32),
            pltpu.VMEM((CHUNK_ROWS, D), jnp.float32),
            pltpu.SemaphoreType.DMA,
            pltpu.SemaphoreType.DMA,
            pltpu.SemaphoreType.DMA,
            pltpu.SemaphoreType.DMA,
        ],
    )


@jax.jit
def kernel(x, mask, table):
    B, S = x.shape
    n_table, D = table.shape
    xf = jnp.where(mask, x.astype(jnp.int32), 0).reshape(-1)
    mi = mask.astype(jnp.int32)
    cnt = mi.sum(axis=1, dtype=jnp.int32)
    alen = (cnt + jnp.int32(UNROLL - 1)) & jnp.int32(-UNROLL)
    ag = alen.reshape(-1, CHUNK_ROWS)
    starts = (jnp.cumsum(ag, axis=1) - ag).reshape(-1)
    ends = starts + alen
    excl = jnp.cumsum(mi, axis=1) - mi
    off = jnp.int32(1) - mi
    off_rank = jnp.cumsum(off, axis=1) - off
    pad_dest = jnp.where(
        off_rank < (alen - cnt)[:, None],
        (starts + cnt)[:, None] + off_rank,
        jnp.int32(TRASH) + (jnp.arange(S, dtype=jnp.int32) % LANES)[None, :],
    )
    dest = jnp.where(mask, starts[:, None] + excl, pad_dest)
    meta = jnp.stack([starts, ends, cnt], axis=-1)
    meta = jnp.pad(meta, ((0, 0), (0, LANES - 3)))
    return _build(B, S, D, n_table)(
        xf, dest.reshape(-1), meta.reshape(-1), table
    )


# speedup vs baseline: 2.4516x; 2.4516x over previous
import jax
import jax.numpy as jnp
from jax import lax
from jax.experimental import pallas as pl
from jax.experimental.pallas import tpu as pltpu
from jax.experimental.pallas import tpu_sc as plsc

NUM_WORKERS = 32
CHUNK_ROWS = 4
LANES = 16


def _build(B, S, D, n_table):
    assert B % NUM_WORKERS == 0
    rows_per_w = B // NUM_WORKERS
    assert rows_per_w % (2 * CHUNK_ROWS) == 0
    n_chunks = rows_per_w // CHUNK_ROWS
    CS = CHUNK_ROWS * S
    assert D % LANES == 0
    d_regs = D // LANES
    n_full_groups = S // LANES
    tail = S - n_full_groups * LANES
    slices = []
    off = 0
    while off < CS:
        n = min(128, CS - off)
        slices.append((off, n))
        off += n

    mesh = plsc.VectorSubcoreMesh(core_axis_name="c", subcore_axis_name="s")

    def body(x_hbm, m_hbm, table_hbm, out_hbm,
             xv0, mv0, rows0, xv1, mv1, rows1, outb, gsem0, gsem1):
        wid = lax.axis_index("s") * 2 + lax.axis_index("c")
        bufs = ((xv0, mv0, rows0, gsem0), (xv1, mv1, rows1, gsem1))

        def load_idx(c, buf):
            xv, mv, _, _ = buf
            base = (wid * rows_per_w + c * CHUNK_ROWS) * S
            pltpu.sync_copy(x_hbm.at[pl.ds(base, CS)], xv)
            pltpu.sync_copy(m_hbm.at[pl.ds(base, CS)], mv.at[pl.ds(0, CS)])

        def gather_copies(buf):
            xv, _, rows_v, gsem = buf
            for off, n in slices:
                yield pltpu.make_async_copy(
                    table_hbm.at[xv.at[pl.ds(off, n)]],
                    rows_v.at[pl.ds(off, n)],
                    gsem,
                )

        def fire(buf):
            for cp in gather_copies(buf):
                cp.start()

        def wait(buf):
            for cp in gather_copies(buf):
                cp.wait()

        def process(c, buf):
            _, mv, rows_v, _ = buf
            row0 = wid * rows_per_w + c * CHUNK_ROWS
            for r in range(CHUNK_ROWS):
                rb = r * S

                def accum_rows(base, mvec, nrows, accs, cnt):
                    out = list(accs)
                    for j in range(nrows):
                        mj = mvec[j]
                        cnt = cnt + mj
                        m = jnp.full((LANES,), mj, jnp.float32)
                        for d in range(d_regs):
                            out[d] = out[d] + rows_v[base + j, pl.ds(d * LANES, LANES)] * m
                    return tuple(out), cnt

                def gbody(g, ac):
                    accs, cnt = ac
                    base = rb + g * LANES
                    mvec = mv[pl.ds(base, LANES)]
                    return accum_rows(base, mvec, LANES, accs, cnt)

                z = jnp.zeros((LANES,), jnp.float32)
                accs, cnt = lax.fori_loop(
                    0, n_full_groups, gbody, ((z,) * d_regs, jnp.float32(0.0))
                )
                if tail:
                    tbase = rb + n_full_groups * LANES
                    mvec = mv[pl.ds(tbase, LANES)]
                    accs, cnt = accum_rows(tbase, mvec, tail, accs, cnt)
                inv = jnp.float32(1.0) / (
                    jnp.full((LANES,), cnt, jnp.float32) + jnp.float32(1e-9)
                )
                for d in range(d_regs):
                    outb[r, pl.ds(d * LANES, LANES)] = accs[d] * inv

            pltpu.sync_copy(outb, out_hbm.at[pl.ds(row0, CHUNK_ROWS)])

        load_idx(0, bufs[0])
        fire(bufs[0])

        def pair_body(i, carry):
            c0 = 2 * i
            load_idx(c0 + 1, bufs[1])
            fire(bufs[1])
            wait(bufs[0])
            process(c0, bufs[0])

            @pl.when(c0 + 2 < n_chunks)
            def _():
                load_idx(c0 + 2, bufs[0])
                fire(bufs[0])

            wait(bufs[1])
            process(c0 + 1, bufs[1])
            return carry

        lax.fori_loop(0, n_chunks // 2, pair_body, 0)

    return pl.kernel(
        body,
        out_type=jax.ShapeDtypeStruct((B, D), jnp.float32),
        mesh=mesh,
        compiler_params=pltpu.CompilerParams(use_tc_tiling_on_sc=False),
        scratch_types=[
            pltpu.VMEM((CS,), jnp.int32),
            pltpu.VMEM((CS + LANES,), jnp.float32),
            pltpu.VMEM((CS, D), jnp.float32),
            pltpu.VMEM((CS,), jnp.int32),
            pltpu.VMEM((CS + LANES,), jnp.float32),
            pltpu.VMEM((CS, D), jnp.float32),
            pltpu.VMEM((CHUNK_ROWS, D), jnp.float32),
            pltpu.SemaphoreType.DMA,
            pltpu.SemaphoreType.DMA,
        ],
    )


@jax.jit
def kernel(x, mask, table):
    B, S = x.shape
    n_table, D = table.shape
    xf = x.reshape(-1).astype(jnp.int32)
    mf = mask.reshape(-1).astype(jnp.float32)
    return _build(B, S, D, n_table)(xf, mf, table)
